# Initial kernel scaffold; baseline (speedup 1.0000x reference)
import jax
import jax.numpy as jnp
from jax.experimental import pallas as pl


def kernel(x):
    raise NotImplementedError("write your pallas kernel here")


# speedup vs baseline: 24.2743x; 24.2743x over previous
import jax
import jax.numpy as jnp
from jax.experimental import pallas as pl


_NEWTON_ITERS = 12


def _body(x_ref, o_ref):
    xb = x_ref[...]
    m = jnp.max(xb, axis=1, keepdims=True)
    tau0 = m - 1.0

    def it(_, tau):
        act = xb > tau
        s = jnp.sum(jnp.where(act, xb, 0.0), axis=1, keepdims=True)
        k = jnp.sum(act.astype(jnp.float32), axis=1, keepdims=True)
        return (s - 1.0) / k

    tau = jax.lax.fori_loop(0, _NEWTON_ITERS, it, tau0)
    o_ref[...] = jnp.maximum(xb - tau, 0.0)


def kernel(x):
    b, n = x.shape
    blk = 8
    return pl.pallas_call(
        _body,
        grid=(b // blk,),
        in_specs=[pl.BlockSpec((blk, n), lambda i: (i, 0))],
        out_specs=pl.BlockSpec((blk, n), lambda i: (i, 0)),
        out_shape=jax.ShapeDtypeStruct((b, n), x.dtype),
    )(x)
